# baseline (speedup 1.0000x reference)
import functools

import jax
import jax.numpy as jnp
from jax import lax
from jax.experimental import pallas as pl
from jax.experimental.pallas import tpu as pltpu
from jax.experimental.pallas import tpu_sc as plsc

_D = 256
_K = 512
_T = 2048
_H = 2
_TS = _T // _H
_N = 32768


_info = plsc.get_sparse_core_info()
_NW = _info.num_cores * _info.num_subcores
_BPW = _N // _NW
_CH = 256
_NCH = _BPW // _CH

_mesh = plsc.VectorSubcoreMesh(core_axis_name="c", subcore_axis_name="s")


@functools.partial(
    pl.kernel, mesh=_mesh,
    out_type=jax.ShapeDtypeStruct((_N, _D), jnp.float32),
    scratch_types=[
        pltpu.VMEM((_CH,), jnp.int32),
        pltpu.VMEM((_CH, _D), jnp.float32),
        pltpu.SemaphoreType.DMA,
    ],
)
def _sc_gather(table_hbm, idx_hbm, out_hbm, idx_v, rows_v, sem):
    wid = lax.axis_index("s") * _info.num_cores + lax.axis_index("c")
    base = wid * _BPW
    for j in range(_NCH):
        off = base + j * _CH
        pltpu.sync_copy(idx_hbm.at[pl.ds(off, _CH)], idx_v)
        pltpu.async_copy(table_hbm.at[idx_v], rows_v, sem).wait()
        pltpu.sync_copy(rows_v, out_hbm.at[pl.ds(off, _CH)])


def _dot(a, b):
    return jax.lax.dot_general(a, b, (((1,), (0,)), ((), ())),
                               preferred_element_type=jnp.float32)


def _dist_block(r, cbt, cbsq, iota):
    rsq = jnp.sum(r * r, axis=1, keepdims=True)
    sc = _dot(r, cbt)
    d = (rsq - 2.0 * sc) + cbsq
    dmin = jnp.min(d, axis=1, keepdims=True)
    idx = jnp.min(jnp.where(d == dmin, iota, _K), axis=1, keepdims=True)
    return idx


def _kd0_kernel(x_ref, cbt_ref, idx_ref):
    iota = jax.lax.broadcasted_iota(jnp.int32, (_TS, _K), 1)
    cbt = cbt_ref[...]
    cbsq = jnp.sum(cbt * cbt, axis=0, keepdims=True)
    for h in range(_H):
        r = x_ref[h * _TS:(h + 1) * _TS, :]
        idx_ref[h * _TS:(h + 1) * _TS, :] = _dist_block(r, cbt, cbsq, iota)


def _kdmid_kernel(with_loss, rp_ref, q_ref, cbt_ref, r_ref, idx_ref, *rest):
    step = pl.program_id(0)
    iota = jax.lax.broadcasted_iota(jnp.int32, (_TS, _K), 1)
    cbt = cbt_ref[...]
    cbsq = jnp.sum(cbt * cbt, axis=0, keepdims=True)
    parts = []
    for h in range(_H):
        sl = slice(h * _TS, (h + 1) * _TS)
        rp = rp_ref[sl, :]
        q = q_ref[sl, :]
        if with_loss:
            parts.append(jnp.sum((rp - q) ** 2))
        x_hat = rp + (q - rp)
        r = rp - x_hat
        r_ref[sl, :] = r
        idx_ref[sl, :] = _dist_block(r, cbt, cbsq, iota)
    if with_loss:
        loss_ref = rest[0]
        part = sum(parts).reshape(1, 1)

        @pl.when(step == 0)
        def _init():
            loss_ref[...] = part

        @pl.when(step != 0)
        def _acc():
            loss_ref[...] += part


def _kd3f_kernel(rp_ref, q_ref, x_ref, cbt_ref, hi3_ref, wt_ref, b_ref,
                 y_ref, p3_ref):
    step = pl.program_id(0)

    @pl.when(step == 0)
    def _make_p3():
        wtb = wt_ref[...].astype(jnp.bfloat16)
        p3_ref[...] = _dot(hi3_ref[...], wtb).astype(jnp.bfloat16)

    iota = jax.lax.broadcasted_iota(jnp.int32, (_TS, _K), 1)
    cbt = cbt_ref[...]
    cbsq = jnp.sum(cbt * cbt, axis=0, keepdims=True)
    for h in range(_H):
        sl = slice(h * _TS, (h + 1) * _TS)
        rp = rp_ref[sl, :]
        q = q_ref[sl, :]
        x_hat = rp + (q - rp)
        r3 = rp - x_hat
        idx = _dist_block(r3, cbt, cbsq, iota)
        onehot = (iota == idx).astype(jnp.bfloat16)
        y_partial = x_ref[sl, :] - r3
        y_ref[sl, :] = (_dot(y_partial, wt_ref[...])
                        + _dot(onehot, p3_ref[...]) + b_ref[...])


def _row_spec():
    return pl.BlockSpec((_T, _D), lambda i: (i, 0))


def kernel(x, codebooks, W, b):
    n = x.shape[0]
    x = x.reshape(n, _D)
    cbt = jnp.transpose(codebooks, (0, 2, 1))
    mask = jnp.uint32(0xFFFF0000)
    bits = jax.lax.bitcast_convert_type(codebooks[3], jnp.uint32)
    hi3 = jax.lax.bitcast_convert_type(bits & mask, jnp.float32) \
        .astype(jnp.bfloat16)
    wt = W.T
    b2 = b.reshape(1, _D)

    cbt_spec = pl.BlockSpec((_D, _K), lambda i: (0, 0))
    idx_spec = pl.BlockSpec((_T, 1), lambda i: (i, 0))

    idx0 = pl.pallas_call(
        _kd0_kernel,
        grid=(n // _T,),
        in_specs=[_row_spec(), cbt_spec],
        out_specs=idx_spec,
        out_shape=jax.ShapeDtypeStruct((n, 1), jnp.int32),
        compiler_params=pltpu.CompilerParams(
            dimension_semantics=("arbitrary",)),
    )(x, cbt[0])

    q0 = _sc_gather(codebooks[0], idx0.reshape(n))

    r1, idx1, loss_sum = pl.pallas_call(
        functools.partial(_kdmid_kernel, True),
        grid=(n // _T,),
        in_specs=[_row_spec(), _row_spec(), cbt_spec],
        out_specs=[_row_spec(), idx_spec, pl.BlockSpec((1, 1), lambda i: (0, 0))],
        out_shape=[jax.ShapeDtypeStruct((n, _D), jnp.float32),
                   jax.ShapeDtypeStruct((n, 1), jnp.int32),
                   jax.ShapeDtypeStruct((1, 1), jnp.float32)],
        compiler_params=pltpu.CompilerParams(
            dimension_semantics=("arbitrary",)),
    )(x, q0, cbt[1])

    q1 = _sc_gather(codebooks[1], idx1.reshape(n))

    r2, idx2 = pl.pallas_call(
        functools.partial(_kdmid_kernel, False),
        grid=(n // _T,),
        in_specs=[_row_spec(), _row_spec(), cbt_spec],
        out_specs=[_row_spec(), idx_spec],
        out_shape=[jax.ShapeDtypeStruct((n, _D), jnp.float32),
                   jax.ShapeDtypeStruct((n, 1), jnp.int32)],
        compiler_params=pltpu.CompilerParams(
            dimension_semantics=("arbitrary",)),
    )(r1, q1, cbt[2])

    q2 = _sc_gather(codebooks[2], idx2.reshape(n))

    y = pl.pallas_call(
        _kd3f_kernel,
        grid=(n // _T,),
        in_specs=[_row_spec(), _row_spec(), _row_spec(), cbt_spec,
                  pl.BlockSpec((_K, _D), lambda i: (0, 0)),
                  pl.BlockSpec((_D, _D), lambda i: (0, 0)),
                  pl.BlockSpec((1, _D), lambda i: (0, 0))],
        out_specs=_row_spec(),
        out_shape=jax.ShapeDtypeStruct((n, _D), jnp.float32),
        scratch_shapes=[pltpu.VMEM((_K, _D), jnp.bfloat16)],
        compiler_params=pltpu.CompilerParams(
            dimension_semantics=("arbitrary",)),
    )(r2, q2, x, cbt[3], hi3, wt, b2)

    y = y.reshape(n, 1, _D)
    idx_out = idx0.reshape(n)
    commit = (loss_sum / (n * _D)).reshape(())
    return y, idx_out, commit


# speedup vs baseline: 2.2674x; 2.2674x over previous
import functools

import jax
import jax.numpy as jnp
from jax.experimental import pallas as pl
from jax.experimental.pallas import tpu as pltpu

_D = 256
_K = 512
_NVQ = 4
_T = 2048
_H = 2
_TS = _T // _H


def _rvq_kernel(x_ref, cbt_ref, hi_ref, mid_ref, lo_ref, wt_ref, b_ref,
                y_ref, idx_ref, loss_ref, p3_ref, cbsq_ref):
    step = pl.program_id(0)

    @pl.when(step == 0)
    def _precompute():
        wtb = wt_ref[...].astype(jnp.bfloat16)
        p3_ref[...] = jax.lax.dot_general(
            hi_ref[3], wtb, (((1,), (0,)), ((), ())),
            preferred_element_type=jnp.float32).astype(jnp.bfloat16)
        for s in range(_NVQ):
            cbt = cbt_ref[s]
            cbsq_ref[s:s + 1, :] = jnp.sum(cbt * cbt, axis=0, keepdims=True)

    iota = jax.lax.broadcasted_iota(jnp.int32, (_TS, _K), 1)
    dot = lambda a, b: jax.lax.dot_general(
        a, b, (((1,), (0,)), ((), ())), preferred_element_type=jnp.float32)

    xs = [x_ref[h * _TS:(h + 1) * _TS, :] for h in range(_H)]
    rs = list(xs)
    ys = [jnp.zeros((_TS, _D), jnp.float32) for _ in range(_H)]
    parts = []

    for s in range(_NVQ):
        cbt = cbt_ref[s]
        cbsq = cbsq_ref[s:s + 1, :]
        for h in range(_H):
            r = rs[h]
            rsq = jnp.sum(r * r, axis=1, keepdims=True)
            sc = dot(r, cbt)
            d = (rsq - 2.0 * sc) + cbsq
            dmin = jnp.min(d, axis=1, keepdims=True)
            idx = jnp.min(jnp.where(d == dmin, iota, _K),
                          axis=1, keepdims=True)
            onehot = (iota == idx).astype(jnp.bfloat16)
            if s == 3:
                out = (dot(ys[h], wt_ref[...])
                       + dot(onehot, p3_ref[...]) + b_ref[...])
                y_ref[h * _TS:(h + 1) * _TS, :] = out
                continue
            q = ((dot(onehot, hi_ref[s]) + dot(onehot, mid_ref[s]))
                 + dot(onehot, lo_ref[s]))
            if s == 0:
                idx_ref[h * _TS:(h + 1) * _TS, :] = idx
                parts.append(jnp.sum((xs[h] - q) ** 2))
            x_hat = r + (q - r)
            rs[h] = r - x_hat
            ys[h] = ys[h] + x_hat

    part = sum(parts).reshape(1, 1)

    @pl.when(step == 0)
    def _init():
        loss_ref[...] = part

    @pl.when(step != 0)
    def _acc():
        loss_ref[...] += part


@functools.partial(jax.jit, static_argnames=())
def kernel(x, codebooks, W, b):
    n = x.shape[0]
    x = x.reshape(n, _D)
    grid = n // _T

    cbt = jnp.transpose(codebooks, (0, 2, 1))
    mask = jnp.uint32(0xFFFF0000)
    bits = jax.lax.bitcast_convert_type(codebooks, jnp.uint32)
    hi32 = jax.lax.bitcast_convert_type(bits & mask, jnp.float32)
    r1 = codebooks - hi32
    r1bits = jax.lax.bitcast_convert_type(r1, jnp.uint32)
    mid32 = jax.lax.bitcast_convert_type(r1bits & mask, jnp.float32)
    hi = hi32.astype(jnp.bfloat16)
    mid = mid32.astype(jnp.bfloat16)
    lo = (r1 - mid32).astype(jnp.bfloat16)
    wt = W.T
    b2 = b.reshape(1, _D)

    full = lambda *_: tuple(0 for _ in range(3))
    y, idx, loss_sum = pl.pallas_call(
        _rvq_kernel,
        grid=(grid,),
        in_specs=[
            pl.BlockSpec((_T, _D), lambda i: (i, 0)),
            pl.BlockSpec((_NVQ, _D, _K), full),
            pl.BlockSpec((_NVQ, _K, _D), full),
            pl.BlockSpec((_NVQ, _K, _D), full),
            pl.BlockSpec((_NVQ, _K, _D), full),
            pl.BlockSpec((_D, _D), lambda i: (0, 0)),
            pl.BlockSpec((1, _D), lambda i: (0, 0)),
        ],
        out_specs=[
            pl.BlockSpec((_T, _D), lambda i: (i, 0)),
            pl.BlockSpec((_T, 1), lambda i: (i, 0)),
            pl.BlockSpec((1, 1), lambda i: (0, 0)),
        ],
        out_shape=[
            jax.ShapeDtypeStruct((n, _D), jnp.float32),
            jax.ShapeDtypeStruct((n, 1), jnp.int32),
            jax.ShapeDtypeStruct((1, 1), jnp.float32),
        ],
        scratch_shapes=[pltpu.VMEM((_K, _D), jnp.bfloat16),
                        pltpu.VMEM((8, _K), jnp.float32)],
        compiler_params=pltpu.CompilerParams(
            dimension_semantics=("arbitrary",)),
    )(x, cbt, hi, mid, lo, wt, b2)

    y = y.reshape(n, 1, _D)
    idx = idx.reshape(n)
    commit = (loss_sum / (n * _D)).reshape(())
    return y, idx, commit
